# baseline (speedup 1.0000x reference)
import jax
import jax.numpy as jnp
from jax.experimental import pallas as pl

_K = 9
_D = 2
_KD = _K * _D
_ROWS_PER_BLOCK = 4000


def _slice_kernel(in_ref, out_ref):
    x = in_ref[...]
    cols = [x[:, 2 * k : 2 * k + 1] for k in range(_K)]
    out_ref[...] = jnp.concatenate(cols, axis=1)


def kernel(edge_index):
    two, b, n, kd = edge_index.shape
    rows = two * b * n
    flat = edge_index.reshape(rows, kd)
    grid = rows // _ROWS_PER_BLOCK
    out = pl.pallas_call(
        _slice_kernel,
        grid=(grid,),
        in_specs=[pl.BlockSpec((_ROWS_PER_BLOCK, kd), lambda i: (i, 0))],
        out_specs=pl.BlockSpec((_ROWS_PER_BLOCK, _K), lambda i: (i, 0)),
        out_shape=jax.ShapeDtypeStruct((rows, _K), edge_index.dtype),
    )(flat)
    return out.reshape(two, b, n, _K)
